# Initial kernel scaffold; baseline (speedup 1.0000x reference)
import jax
import jax.numpy as jnp
from jax.experimental import pallas as pl


def kernel(id, t, a_table, b_table):
    raise NotImplementedError("write your pallas kernel here")


# speedup vs baseline: 5.9632x; 5.9632x over previous
import functools

import numpy as np
import jax
import jax.numpy as jnp
from jax import lax
from jax.experimental import pallas as pl
from jax.experimental.pallas import tpu as pltpu
from jax.experimental.pallas import tpu_sc as plsc

_PERIOD = np.float32(365.25)
_TWO_PI = np.float32(2.0 * np.pi)
_N = 64
_B = 16384

_NC = 2
_NS = 16
_NW = _NC * _NS
_BPW = _B // _NW
_CHUNK = 128
_NCHUNK = _BPW // _CHUNK

_BT = 1024
_GRID = _B // _BT


def _sc_gather(id2d, a_table, b_table):
    mesh = plsc.VectorSubcoreMesh(core_axis_name="c", subcore_axis_name="s")

    @functools.partial(
        pl.kernel,
        mesh=mesh,
        compiler_params=pltpu.CompilerParams(use_tc_tiling_on_sc=False),
        out_type=[
            jax.ShapeDtypeStruct((_B, _N), jnp.float32),
            jax.ShapeDtypeStruct((_B, _N), jnp.float32),
        ],
        scratch_types=[
            pltpu.VMEM((_NCHUNK, _CHUNK), jnp.int32),
            pltpu.VMEM((_BPW, _N), jnp.float32),
            pltpu.VMEM((_BPW, _N), jnp.float32),
            pltpu.SemaphoreType.DMA,
            pltpu.SemaphoreType.DMA,
        ],
    )
    def k(id_hbm, a_hbm, b_hbm, a_out, b_out, idx_v, arows, brows, sem_a, sem_b):
        wid = lax.axis_index("s") * _NC + lax.axis_index("c")
        base = wid * _BPW
        pltpu.sync_copy(id_hbm.at[pl.ds(wid * _NCHUNK, _NCHUNK)], idx_v)
        copies = []
        for j in range(_NCHUNK):
            rows_sl = pl.ds(j * _CHUNK, _CHUNK)
            copies.append(pltpu.async_copy(
                a_hbm.at[idx_v.at[j]], arows.at[rows_sl], sem_a))
            copies.append(pltpu.async_copy(
                b_hbm.at[idx_v.at[j]], brows.at[rows_sl], sem_b))
        for c in copies:
            c.wait()
        pltpu.sync_copy(arows, a_out.at[pl.ds(base, _BPW)])
        pltpu.sync_copy(brows, b_out.at[pl.ds(base, _BPW)])

    return k(id2d, a_table, b_table)


def _tc_body(t_ref, a_ref, b_ref, atab_ref, btab_ref, o_ref):
    n = (lax.broadcasted_iota(jnp.int32, (_BT, _N), 1) + 1).astype(jnp.float32)
    x = _TWO_PI * (_PERIOD * t_ref[...]) / n
    a0 = atab_ref[0:1, :]
    b0 = btab_ref[0:1, :]
    cos_part = jnp.cos(x) * (a_ref[...] + a0)
    sin_part = jnp.sin(x) * (b_ref[...] + b0)
    o_ref[...] = (jnp.sum(cos_part, axis=1, keepdims=True)
                  + jnp.sum(sin_part, axis=1, keepdims=True))


def _tc_fourier(t, a_rows, b_rows, a_table, b_table):
    return pl.pallas_call(
        _tc_body,
        grid=(_GRID,),
        in_specs=[
            pl.BlockSpec((_BT, 1), lambda i: (i, 0)),
            pl.BlockSpec((_BT, _N), lambda i: (i, 0)),
            pl.BlockSpec((_BT, _N), lambda i: (i, 0)),
            pl.BlockSpec((8, _N), lambda i: (0, 0)),
            pl.BlockSpec((8, _N), lambda i: (0, 0)),
        ],
        out_specs=pl.BlockSpec((_BT, 1), lambda i: (i, 0)),
        out_shape=jax.ShapeDtypeStruct((_B, 1), jnp.float32),
    )(t, a_rows, b_rows, a_table, b_table)


def kernel(id, t, a_table, b_table):
    id2d = id.astype(jnp.int32).reshape(_NW * _NCHUNK, _CHUNK)
    a_rows, b_rows = _sc_gather(id2d, a_table, b_table)
    return _tc_fourier(t, a_rows, b_rows, a_table, b_table)
